# baseline (speedup 1.0000x reference)
import functools

import jax
import jax.numpy as jnp
from jax.experimental import pallas as pl
from jax.experimental.pallas import tpu as pltpu

_B = 64
_K_TOP = 256
_N = 32768
_EPS = 0.1
_MAX_ITER = 200
_PAD = 1e-16
_BC = 64
_WARM = 9


def _body(s_ref, out_ref, m_ref, t_ref, p_scr, sem):
    p = pl.program_id(0)
    i = pl.program_id(1)
    g = pl.num_programs(1)
    n = s_ref.shape[1]
    mu = 1.0 / n
    nu0 = _K_TOP / n
    nu1 = (n - _K_TOP) / n

    @pl.when(p == 0)
    def _phase_max():
        s = s_ref[...]
        local = jnp.max(jnp.maximum(s * s, (s - 1.0) * (s - 1.0)))

        @pl.when(i == 0)
        def _():
            m_ref[0] = local

        @pl.when(i != 0)
        def _():
            m_ref[0] = jnp.maximum(m_ref[0], local)

    @pl.when(p == 1)
    def _phase_sinkhorn():
        s = s_ref[...]
        c = (1.0 / _EPS) / m_ref[0]
        t_ref[...] = jnp.exp((2.0 * c) * s - c)

        one = jnp.ones((_BC, 1), jnp.float32)
        u0 = nu0 * one
        u1 = nu1 * one

        def update(u0, u1):
            w = pl.reciprocal(u0 + u1 * t_ref[...], approx=True)
            s0 = jnp.sum(w, axis=1, keepdims=True)
            r0 = mu * s0
            r1 = mu * (n - u0 * s0) / u1
            return nu0 / (r0 + _PAD), nu1 / (r1 + _PAD)

        u0, u1 = jax.lax.fori_loop(0, _WARM, lambda k, c_: update(*c_), (u0, u1))

        def cond(carry):
            it, _, _, changed = carry
            return jnp.logical_and(it < _MAX_ITER - 1 - _WARM, changed)

        def body(carry):
            it, u0, u1, _ = carry
            n0, n1 = update(u0, u1)
            changed = jnp.logical_or(
                jnp.any(jnp.abs(n0 - u0) > 1e-6 * u0),
                jnp.any(jnp.abs(n1 - u1) > 1e-6 * u1),
            )
            return it + 1, n0, n1, changed

        _, u0, u1, _ = jax.lax.while_loop(
            cond, body, (jnp.int32(0), u0, u1, jnp.bool_(True))
        )

        tt = t_ref[...]
        w = 1.0 / (u0 + u1 * tt)
        p_scr[i, :, :n] = (mu * u0) * w
        pltpu.make_async_copy(
            p_scr.at[i, :, pl.ds(0, n)],
            out_ref.at[pl.ds(i * _BC, _BC), pl.ds(0, n)],
            sem,
        ).start()
        p_scr[i, :, n:] = (mu * u1) * (tt * w)
        pltpu.make_async_copy(
            p_scr.at[i, :, pl.ds(n, n)],
            out_ref.at[pl.ds(i * _BC, _BC), pl.ds(n, n)],
            sem,
        ).start()

        @pl.when(i == g - 1)
        def _drain():
            for j in range(_B // _BC):
                for h in range(2):
                    pltpu.make_async_copy(
                        p_scr.at[j, :, pl.ds(h * n, n)],
                        out_ref.at[pl.ds(j * _BC, _BC), pl.ds(h * n, n)],
                        sem,
                    ).wait()


@functools.partial(jax.jit, static_argnames=())
def kernel(scores):
    b, n = scores.shape
    out = pl.pallas_call(
        _body,
        grid=(2, b // _BC),
        in_specs=[
            pl.BlockSpec((_BC, n), lambda p, i: (i, 0)),
        ],
        out_specs=pl.BlockSpec(memory_space=pl.ANY),
        out_shape=jax.ShapeDtypeStruct((b, 2 * n), jnp.float32),
        scratch_shapes=[
            pltpu.SMEM((1,), jnp.float32),
            pltpu.VMEM((_BC, n), jnp.float32),
            pltpu.VMEM((b // _BC, _BC, 2 * n), jnp.float32),
            pltpu.SemaphoreType.DMA,
        ],
    )(scores)
    return out.reshape(b, 2, n)


# speedup vs baseline: 1.0103x; 1.0103x over previous
import functools

import jax
import jax.numpy as jnp
from jax.experimental import pallas as pl
from jax.experimental.pallas import tpu as pltpu

_B = 64
_K_TOP = 256
_N = 32768
_EPS = 0.1
_MAX_ITER = 200
_PAD = 1e-16
_BC = 32
_WARM = 9


def _body(s_ref, out_ref, m_ref, t_ref, p_scr, sem):
    p = pl.program_id(0)
    i = pl.program_id(1)
    g = pl.num_programs(1)
    n = s_ref.shape[1]
    mu = 1.0 / n
    nu0 = _K_TOP / n
    nu1 = (n - _K_TOP) / n

    @pl.when(p == 0)
    def _phase_max():
        s = s_ref[...]
        local = jnp.max(jnp.maximum(s * s, (s - 1.0) * (s - 1.0)))

        @pl.when(i == 0)
        def _():
            m_ref[0] = local

        @pl.when(i != 0)
        def _():
            m_ref[0] = jnp.maximum(m_ref[0], local)

    @pl.when(p == 1)
    def _phase_sinkhorn():
        s = s_ref[...]
        c = (1.0 / _EPS) / m_ref[0]
        t_ref[...] = jnp.exp((2.0 * c) * s - c)

        one = jnp.ones((_BC, 1), jnp.float32)
        u0 = nu0 * one
        u1 = nu1 * one

        def update(u0, u1):
            w = pl.reciprocal(u0 + u1 * t_ref[...], approx=True, full_range=False)
            s0 = jnp.sum(w, axis=1, keepdims=True)
            r0 = mu * s0
            r1 = mu * (n - u0 * s0) / u1
            return nu0 / (r0 + _PAD), nu1 / (r1 + _PAD)

        u0, u1 = jax.lax.fori_loop(0, _WARM, lambda k, c_: update(*c_), (u0, u1))

        def cond(carry):
            it, _, _, changed = carry
            return jnp.logical_and(it < _MAX_ITER - 1 - _WARM, changed)

        def body(carry):
            it, u0, u1, _ = carry
            n0, n1 = update(u0, u1)
            changed = jnp.logical_or(
                jnp.any(jnp.abs(n0 - u0) > 1e-6 * u0),
                jnp.any(jnp.abs(n1 - u1) > 1e-6 * u1),
            )
            return it + 1, n0, n1, changed

        _, u0, u1, _ = jax.lax.while_loop(
            cond, body, (jnp.int32(0), u0, u1, jnp.bool_(True))
        )

        tt = t_ref[...]
        w = 1.0 / (u0 + u1 * tt)
        p_scr[i, :, :n] = (mu * u0) * w
        pltpu.make_async_copy(
            p_scr.at[i, :, pl.ds(0, n)],
            out_ref.at[pl.ds(i * _BC, _BC), pl.ds(0, n)],
            sem,
        ).start()
        p_scr[i, :, n:] = (mu * u1) * (tt * w)
        pltpu.make_async_copy(
            p_scr.at[i, :, pl.ds(n, n)],
            out_ref.at[pl.ds(i * _BC, _BC), pl.ds(n, n)],
            sem,
        ).start()

        @pl.when(i == g - 1)
        def _drain():
            for j in range(_B // _BC):
                for h in range(2):
                    pltpu.make_async_copy(
                        p_scr.at[j, :, pl.ds(h * n, n)],
                        out_ref.at[pl.ds(j * _BC, _BC), pl.ds(h * n, n)],
                        sem,
                    ).wait()


@functools.partial(jax.jit, static_argnames=())
def kernel(scores):
    b, n = scores.shape
    out = pl.pallas_call(
        _body,
        grid=(2, b // _BC),
        in_specs=[
            pl.BlockSpec((_BC, n), lambda p, i: (i, 0)),
        ],
        out_specs=pl.BlockSpec(memory_space=pl.ANY),
        out_shape=jax.ShapeDtypeStruct((b, 2 * n), jnp.float32),
        scratch_shapes=[
            pltpu.SMEM((1,), jnp.float32),
            pltpu.VMEM((_BC, n), jnp.float32),
            pltpu.VMEM((b // _BC, _BC, 2 * n), jnp.float32),
            pltpu.SemaphoreType.DMA,
        ],
    )(scores)
    return out.reshape(b, 2, n)


# speedup vs baseline: 1.0112x; 1.0008x over previous
import functools

import jax
import jax.numpy as jnp
from jax.experimental import pallas as pl
from jax.experimental.pallas import tpu as pltpu

_B = 64
_K_TOP = 256
_N = 32768
_EPS = 0.1
_MAX_ITER = 200
_PAD = 1e-16
_BC = 32
_WARM = 9


def _body(s_ref, out_ref, m_ref, t_ref, p_scr, sem):
    p = pl.program_id(0)
    i = pl.program_id(1)
    g = pl.num_programs(1)
    n = s_ref.shape[1]
    mu = 1.0 / n
    nu0 = _K_TOP / n
    nu1 = (n - _K_TOP) / n

    @pl.when(p == 0)
    def _phase_max():
        s = s_ref[...]
        local = jnp.max(jnp.maximum(s * s, (s - 1.0) * (s - 1.0)))

        @pl.when(i == 0)
        def _():
            m_ref[0] = local

        @pl.when(i != 0)
        def _():
            m_ref[0] = jnp.maximum(m_ref[0], local)

    @pl.when(p == 1)
    def _phase_sinkhorn():
        s = s_ref[...]
        c = (1.0 / _EPS) / m_ref[0]
        t_ref[...] = jnp.exp((2.0 * c) * s - c)

        one = jnp.ones((_BC, 1), jnp.float32)
        u0 = nu0 * one
        u1 = nu1 * one

        def update(u0, u1):
            w = pl.reciprocal(u0 + u1 * t_ref[...], approx=True)
            s0 = jnp.sum(w, axis=1, keepdims=True)
            r0 = mu * s0
            r1 = mu * (n - u0 * s0) / u1
            return nu0 / (r0 + _PAD), nu1 / (r1 + _PAD)

        u0, u1 = jax.lax.fori_loop(0, _WARM, lambda k, c_: update(*c_), (u0, u1))

        def cond(carry):
            it, _, _, changed = carry
            return jnp.logical_and(it < _MAX_ITER - 1 - _WARM, changed)

        def body(carry):
            it, u0, u1, _ = carry
            n0, n1 = update(u0, u1)
            changed = jnp.logical_or(
                jnp.any(jnp.abs(n0 - u0) > 1e-6 * u0),
                jnp.any(jnp.abs(n1 - u1) > 1e-6 * u1),
            )
            return it + 1, n0, n1, changed

        _, u0, u1, _ = jax.lax.while_loop(
            cond, body, (jnp.int32(0), u0, u1, jnp.bool_(True))
        )

        tt = t_ref[...]
        w = 1.0 / (u0 + u1 * tt)
        p_scr[i, :, :n] = (mu * u0) * w
        pltpu.make_async_copy(
            p_scr.at[i, :, pl.ds(0, n)],
            out_ref.at[pl.ds(i * _BC, _BC), pl.ds(0, n)],
            sem,
        ).start()
        p_scr[i, :, n:] = (mu * u1) * (tt * w)
        pltpu.make_async_copy(
            p_scr.at[i, :, pl.ds(n, n)],
            out_ref.at[pl.ds(i * _BC, _BC), pl.ds(n, n)],
            sem,
        ).start()

        @pl.when(i == g - 1)
        def _drain():
            for j in range(_B // _BC):
                for h in range(2):
                    pltpu.make_async_copy(
                        p_scr.at[j, :, pl.ds(h * n, n)],
                        out_ref.at[pl.ds(j * _BC, _BC), pl.ds(h * n, n)],
                        sem,
                    ).wait()


@functools.partial(jax.jit, static_argnames=())
def kernel(scores):
    b, n = scores.shape
    out = pl.pallas_call(
        _body,
        grid=(2, b // _BC),
        in_specs=[
            pl.BlockSpec((_BC, n), lambda p, i: (i, 0)),
        ],
        out_specs=pl.BlockSpec(memory_space=pl.ANY),
        out_shape=jax.ShapeDtypeStruct((b, 2 * n), jnp.float32),
        scratch_shapes=[
            pltpu.SMEM((1,), jnp.float32),
            pltpu.VMEM((_BC, n), jnp.float32),
            pltpu.VMEM((b // _BC, _BC, 2 * n), jnp.float32),
            pltpu.SemaphoreType.DMA,
        ],
    )(scores)
    return out.reshape(b, 2, n)
